# baseline (device time: 342347 ns/iter reference)
import jax
import jax.numpy as jnp
from jax import lax
from jax.experimental import pallas as pl
from jax.experimental.pallas import tpu as pltpu

N_DEV = 8
Q = 4
COMM_DTYPE = jnp.bfloat16
IN_DTYPE = jnp.float8_e4m3fn


def kernel(x, w_mat, scale_x, scale_w):
    m, k_shard = x.shape
    _, n = w_mat.shape
    m_per = m // N_DEV
    nq = n // (2 * Q)
    nw = n // 4

    def body(x_ref, w_ref, s_ref, out_ref,
             x8, w8, xst, wst, ost,
             sem_x, sem_w, sem_o,
             comm_a, comm_b, send_a, recv_a, send_b, recv_b):
        my = lax.axis_index("i")
        left = lax.rem(my + N_DEV - 1, N_DEV)
        right = lax.rem(my + 1, N_DEV)

        def col(d, q):
            return (d * Q + q) * nq

        def partial(c, d, q):
            xa = x8[pl.ds(c * m_per, m_per), :]
            return lax.dot_general(
                xa, w8[:, col(d, q):col(d, q) + nq],
                (((1,), (0,)), ((), ())),
                preferred_element_type=jnp.float32,
            )

        def mk(h, d, q):
            ss = h % 2
            rs = (h + 1) % 2
            comm = comm_a if d == 0 else comm_b
            send = send_a if d == 0 else send_b
            recv = recv_a if d == 0 else recv_b
            tgt = right if d == 0 else left
            return pltpu.make_async_remote_copy(
                src_ref=comm.at[ss, q], dst_ref=comm.at[rs, q],
                send_sem=send.at[ss, q], recv_sem=recv.at[rs, q],
                device_id=(tgt,), device_id_type=pl.DeviceIdType.MESH,
            )

        cp_x = pltpu.make_async_copy(x_ref, xst, sem_x)
        cp_x.start()
        wblocks = [0, 2, 1, 3]

        def w_cp(i):
            b = wblocks[i]
            return pltpu.make_async_copy(
                w_ref.at[:, b * nw:(b + 1) * nw], wst.at[i % 2],
                sem_w.at[i % 2])

        wcps = [w_cp(i) for i in range(4)]
        wcps[0].start()
        wcps[1].start()
        cp_x.wait()
        x8[...] = xst[...].astype(IN_DTYPE)

        barrier_sem = pltpu.get_barrier_semaphore()
        for nbr in (left, right):
            pl.semaphore_signal(
                barrier_sem, inc=1,
                device_id=(nbr,), device_id_type=pl.DeviceIdType.MESH,
            )
        pl.semaphore_wait(barrier_sem, 2)

        blk_subs = {0: (0, (0, 1)), 1: (0, (2, 3)),
                    2: (1, (0, 1)), 3: (1, (2, 3))}
        c0 = [lax.rem(my + N_DEV - 1, N_DEV), lax.rem(my + 1, N_DEV)]
        sends = {}
        for i in range(4):
            b = wblocks[i]
            wcps[i].wait()
            w8[:, b * nw:(b + 1) * nw] = wst[i % 2, :, :].astype(IN_DTYPE)
            if i + 2 < 4:
                wcps[i + 2].start()
            d, qs = blk_subs[b]
            comm = comm_a if d == 0 else comm_b
            for q in qs:
                comm[0, q, :, :] = partial(c0[d], d, q).astype(COMM_DTYPE)
                r = mk(0, d, q)
                r.start()
                sends[(0, d, q)] = r

        sc = s_ref[0]
        out_cps = []
        for h in range(N_DEV - 1):
            rs = (h + 1) % 2
            ch = [lax.rem(my + N_DEV - 2 - h, N_DEV),
                  lax.rem(my + 2 + h, N_DEV)]
            for q in range(Q):
                for d in range(2):
                    comm = comm_a if d == 0 else comm_b
                    p = partial(ch[d], d, q)
                    sends[(h, d, q)].wait_recv()
                    acc = p + comm[rs, q, :, :].astype(jnp.float32)
                    if h < N_DEV - 2:
                        comm[rs, q, :, :] = acc.astype(COMM_DTYPE)
                        if h >= 1:
                            sends[(h - 1, d, q)].wait_send()
                        r = mk(h + 1, d, q)
                        r.start()
                        sends[(h + 1, d, q)] = r
                    else:
                        j = len(out_cps)
                        slot = j % 2
                        if j >= 2:
                            out_cps[j - 2].wait()
                        ost[slot, :, :] = jnp.maximum(acc * sc, 0.0)
                        cp = pltpu.make_async_copy(
                            ost.at[slot],
                            out_ref.at[:, col(d, q):col(d, q) + nq],
                            sem_o.at[slot])
                        cp.start()
                        out_cps.append(cp)

        out_cps[-2].wait()
        out_cps[-1].wait()
        for q in range(Q):
            for d in range(2):
                sends[(N_DEV - 3, d, q)].wait_send()
                sends[(N_DEV - 2, d, q)].wait_send()

    s = (scale_x * scale_w).astype(jnp.float32)
    return pl.pallas_call(
        body,
        out_shape=jax.ShapeDtypeStruct((m_per, n), jnp.float32),
        in_specs=[
            pl.BlockSpec(memory_space=pltpu.HBM),
            pl.BlockSpec(memory_space=pltpu.HBM),
            pl.BlockSpec(memory_space=pltpu.SMEM),
        ],
        out_specs=pl.BlockSpec(memory_space=pltpu.HBM),
        scratch_shapes=[
            pltpu.VMEM((m, k_shard), IN_DTYPE),
            pltpu.VMEM((k_shard, n), IN_DTYPE),
            pltpu.VMEM((m, k_shard), jnp.float32),
            pltpu.VMEM((2, k_shard, nw), jnp.float32),
            pltpu.VMEM((2, m_per, nq), jnp.float32),
            pltpu.SemaphoreType.DMA,
            pltpu.SemaphoreType.DMA((2,)),
            pltpu.SemaphoreType.DMA((2,)),
            pltpu.VMEM((2, Q, m_per, nq), COMM_DTYPE),
            pltpu.VMEM((2, Q, m_per, nq), COMM_DTYPE),
            pltpu.SemaphoreType.DMA((2, Q)),
            pltpu.SemaphoreType.DMA((2, Q)),
            pltpu.SemaphoreType.DMA((2, Q)),
            pltpu.SemaphoreType.DMA((2, Q)),
        ],
        compiler_params=pltpu.CompilerParams(
            collective_id=0,
            vmem_limit_bytes=100 * 1024 * 1024,
        ),
    )(x, w_mat, s)


# device time: 319780 ns/iter; 1.0706x vs baseline; 1.0706x over previous
import jax
import jax.numpy as jnp
from jax import lax
from jax.experimental import pallas as pl
from jax.experimental.pallas import tpu as pltpu

N_DEV = 8
Q = 4
COMM_DTYPE = jnp.bfloat16
IN_DTYPE = jnp.float8_e4m3fn


def kernel(x, w_mat, scale_x, scale_w):
    m, k_shard = x.shape
    _, n = w_mat.shape
    m_per = m // N_DEV
    nq = n // (2 * Q)
    nw = n // 4

    def body(x_ref, w_ref, s_ref, out_ref,
             x8, w8, xst, wst, ost,
             sem_x, sem_w, sem_o,
             comm_a, comm_b, send_a, recv_a, send_b, recv_b,
             c8s_a, c8r_a, c8s_b, c8r_b,
             send8_a, recv8_a, send8_b, recv8_b):
        my = lax.axis_index("i")
        left = lax.rem(my + N_DEV - 1, N_DEV)
        right = lax.rem(my + 1, N_DEV)

        def col(d, q):
            return (d * Q + q) * nq

        def partial(c, d, q):
            xa = x8[pl.ds(c * m_per, m_per), :]
            return lax.dot_general(
                xa, w8[:, col(d, q):col(d, q) + nq],
                (((1,), (0,)), ((), ())),
                preferred_element_type=jnp.float32,
            )

        def mk(h, d, q):
            ss = h % 2
            rs = (h + 1) % 2
            comm = comm_a if d == 0 else comm_b
            send = send_a if d == 0 else send_b
            recv = recv_a if d == 0 else recv_b
            tgt = right if d == 0 else left
            return pltpu.make_async_remote_copy(
                src_ref=comm.at[ss, q], dst_ref=comm.at[rs, q],
                send_sem=send.at[ss, q], recv_sem=recv.at[rs, q],
                device_id=(tgt,), device_id_type=pl.DeviceIdType.MESH,
            )

        def mk8(d, q):
            c8s = c8s_a if d == 0 else c8s_b
            c8r = c8r_a if d == 0 else c8r_b
            send = send8_a if d == 0 else send8_b
            recv = recv8_a if d == 0 else recv8_b
            tgt = right if d == 0 else left
            return pltpu.make_async_remote_copy(
                src_ref=c8s.at[q], dst_ref=c8r.at[q],
                send_sem=send.at[q], recv_sem=recv.at[q],
                device_id=(tgt,), device_id_type=pl.DeviceIdType.MESH,
            )

        cp_x = pltpu.make_async_copy(x_ref, xst, sem_x)
        cp_x.start()
        wblocks = [0, 2, 1, 3]

        def w_cp(i):
            b = wblocks[i]
            return pltpu.make_async_copy(
                w_ref.at[:, b * nw:(b + 1) * nw], wst.at[i % 2],
                sem_w.at[i % 2])

        wcps = [w_cp(i) for i in range(4)]
        wcps[0].start()
        wcps[1].start()
        cp_x.wait()
        x8[...] = xst[...].astype(IN_DTYPE)

        barrier_sem = pltpu.get_barrier_semaphore()
        for nbr in (left, right):
            pl.semaphore_signal(
                barrier_sem, inc=1,
                device_id=(nbr,), device_id_type=pl.DeviceIdType.MESH,
            )
        pl.semaphore_wait(barrier_sem, 2)

        blk_subs = {0: (0, (0, 1)), 1: (0, (2, 3)),
                    2: (1, (0, 1)), 3: (1, (2, 3))}
        c0 = [lax.rem(my + N_DEV - 1, N_DEV), lax.rem(my + 1, N_DEV)]
        sends = {}
        for i in range(4):
            b = wblocks[i]
            wcps[i].wait()
            w8[:, b * nw:(b + 1) * nw] = wst[i % 2, :, :].astype(IN_DTYPE)
            if i + 2 < 4:
                wcps[i + 2].start()
            d, qs = blk_subs[b]
            c8s = c8s_a if d == 0 else c8s_b
            for q in qs:
                c8s[q, :, :] = partial(c0[d], d, q).astype(IN_DTYPE)
                r = mk8(d, q)
                r.start()
                sends[(0, d, q)] = r

        sc = s_ref[0]
        out_cps = []
        for h in range(N_DEV - 1):
            rs = (h + 1) % 2
            ch = [lax.rem(my + N_DEV - 2 - h, N_DEV),
                  lax.rem(my + 2 + h, N_DEV)]
            for q in range(Q):
                for d in range(2):
                    comm = comm_a if d == 0 else comm_b
                    p = partial(ch[d], d, q)
                    sends[(h, d, q)].wait_recv()
                    if h == 0:
                        c8r = c8r_a if d == 0 else c8r_b
                        rv = c8r[q, :, :].astype(jnp.float32)
                    else:
                        rv = comm[rs, q, :, :].astype(jnp.float32)
                    acc = p + rv
                    if h < N_DEV - 2:
                        comm[rs, q, :, :] = acc.astype(COMM_DTYPE)
                        if h >= 1:
                            sends[(h - 1, d, q)].wait_send()
                        r = mk(h + 1, d, q)
                        r.start()
                        sends[(h + 1, d, q)] = r
                    else:
                        j = len(out_cps)
                        slot = j % 2
                        if j >= 2:
                            out_cps[j - 2].wait()
                        ost[slot, :, :] = jnp.maximum(acc * sc, 0.0)
                        cp = pltpu.make_async_copy(
                            ost.at[slot],
                            out_ref.at[:, col(d, q):col(d, q) + nq],
                            sem_o.at[slot])
                        cp.start()
                        out_cps.append(cp)

        out_cps[-2].wait()
        out_cps[-1].wait()
        for q in range(Q):
            for d in range(2):
                sends[(N_DEV - 3, d, q)].wait_send()
                sends[(N_DEV - 2, d, q)].wait_send()

    s = (scale_x * scale_w).astype(jnp.float32)
    return pl.pallas_call(
        body,
        out_shape=jax.ShapeDtypeStruct((m_per, n), jnp.float32),
        in_specs=[
            pl.BlockSpec(memory_space=pltpu.HBM),
            pl.BlockSpec(memory_space=pltpu.HBM),
            pl.BlockSpec(memory_space=pltpu.SMEM),
        ],
        out_specs=pl.BlockSpec(memory_space=pltpu.HBM),
        scratch_shapes=[
            pltpu.VMEM((m, k_shard), IN_DTYPE),
            pltpu.VMEM((k_shard, n), IN_DTYPE),
            pltpu.VMEM((m, k_shard), jnp.float32),
            pltpu.VMEM((2, k_shard, nw), jnp.float32),
            pltpu.VMEM((2, m_per, nq), jnp.float32),
            pltpu.SemaphoreType.DMA,
            pltpu.SemaphoreType.DMA((2,)),
            pltpu.SemaphoreType.DMA((2,)),
            pltpu.VMEM((2, Q, m_per, nq), COMM_DTYPE),
            pltpu.VMEM((2, Q, m_per, nq), COMM_DTYPE),
            pltpu.SemaphoreType.DMA((2, Q)),
            pltpu.SemaphoreType.DMA((2, Q)),
            pltpu.SemaphoreType.DMA((2, Q)),
            pltpu.SemaphoreType.DMA((2, Q)),
            pltpu.VMEM((Q, m_per, nq), IN_DTYPE),
            pltpu.VMEM((Q, m_per, nq), IN_DTYPE),
            pltpu.VMEM((Q, m_per, nq), IN_DTYPE),
            pltpu.VMEM((Q, m_per, nq), IN_DTYPE),
            pltpu.SemaphoreType.DMA((Q,)),
            pltpu.SemaphoreType.DMA((Q,)),
            pltpu.SemaphoreType.DMA((Q,)),
            pltpu.SemaphoreType.DMA((Q,)),
        ],
        compiler_params=pltpu.CompilerParams(
            collective_id=0,
            vmem_limit_bytes=100 * 1024 * 1024,
        ),
    )(x, w_mat, s)
